# baseline (speedup 1.0000x reference)
import jax
import jax.numpy as jnp
from jax import lax
from jax.experimental import pallas as pl
from jax.experimental.pallas import tpu as pltpu
from jax.experimental.pallas import tpu_sc as plsc

NC, NS, L = 2, 16, 16
NW = NC * NS
GRID = 14
CELLS = GRID * GRID
C = 30
NB = 256
GROUPS = NB // L


def _sq(x):
    return x * x


def _sqrt16(x):
    i = lax.bitcast_convert_type(x, jnp.int32)
    i = jnp.int32(0x5F3759DF) - lax.shift_right_arithmetic(i, 1)
    y = lax.bitcast_convert_type(i, jnp.float32)
    for _ in range(3):
        y = y * (1.5 - 0.5 * x * y * y)
    return x * y


def _group_loss(pv, tv, b0):
    acc4 = []
    for lane in range(4):
        j0 = 10 + lane * 5
        d = pv[j0, pl.ds(b0, L)] - tv[j0, pl.ds(b0, L)]
        a = d * d
        for j in range(j0 + 1, j0 + 5):
            d = pv[j, pl.ds(b0, L)] - tv[j, pl.ds(b0, L)]
            a = a + d * d
        acc4.append(a)
    cls = (acc4[0] + acc4[1]) + (acc4[2] + acc4[3])

    p = [pv[j, pl.ds(b0, L)] for j in range(10)]
    t = [tv[j, pl.ds(b0, L)] for j in range(10)]
    conf = t[4]
    coo = jnp.where(conf > 0, 1.0, 0.0).astype(jnp.float32)
    noo = jnp.where(conf == 0, 1.0, 0.0).astype(jnp.float32)
    noo_row = _sq(p[4] - t[4]) + _sq(p[9] - t[9])

    t_xmin = t[0] - 0.5 * t[2]
    t_ymin = t[1] - 0.5 * t[3]
    t_xmax = t[0] + 0.5 * t[2]
    t_ymax = t[1] + 0.5 * t[3]
    area2 = t[2] * t[3]

    ious = []
    for k in (0, 5):
        xmin = p[k + 0] - 0.5 * p[k + 2]
        ymin = p[k + 1] - 0.5 * p[k + 3]
        xmax = p[k + 0] + 0.5 * p[k + 2]
        ymax = p[k + 1] + 0.5 * p[k + 3]
        ltx = jnp.maximum(xmin, t_xmin)
        lty = jnp.maximum(ymin, t_ymin)
        rbx = jnp.minimum(xmax, t_xmax)
        rby = jnp.minimum(ymax, t_ymax)
        whx = jnp.where(rbx - ltx < 0, 1.0, 0.0).astype(jnp.float32)
        why = jnp.where(rby - lty < 0, 1.0, 0.0).astype(jnp.float32)
        inter = whx * why
        area1 = p[k + 2] * p[k + 3]
        ious.append(inter / (area1 + area2 - inter))
    iou0, iou1 = ious
    isn0 = iou0 != iou0
    isn1 = iou1 != iou1
    sel = (iou1 > iou0) | (isn1 & (~isn0))

    rp = [jnp.where(sel, p[5 + j], p[j]) for j in range(5)]
    rt = [jnp.where(sel, t[5 + j], t[j]) for j in range(5)]

    contain = _sq(rp[4] - rt[4])
    loc = (_sq(rp[0] - rt[0]) + _sq(rp[1] - rt[1])
           + _sq(_sqrt16(rp[2]) - _sqrt16(rt[2]))
           + _sq(_sqrt16(rp[3]) - _sqrt16(rt[3])))
    return coo * (loc + 2.0 * contain + cls) + noo * noo_row


BLK = 128
UNITS = CELLS * (NB // BLK)
ROUNDS = (UNITS + NW - 1) // NW


def _body(pred_hbm, targ_hbm, out_hbm,
          pv0, tv0, pv1, tv1, acc_v, sp0, st0, sp1, st1):
    w = lax.axis_index("s") * NC + lax.axis_index("c")

    def unit_id(r):
        return jnp.minimum(w + NW * r, UNITS - 1)

    def src_refs(h):
        cell = h // 2
        ci = cell // GRID
        cj = cell - ci * GRID
        b0 = (h - cell * 2) * BLK
        return (pred_hbm.at[ci, cj, :, pl.ds(b0, BLK)],
                targ_hbm.at[ci, cj, :, pl.ds(b0, BLK)])

    def issue(r, pv, tv, sp, st):
        ps, ts = src_refs(unit_id(r))
        pltpu.async_copy(ps, pv, sp)
        pltpu.async_copy(ts, tv, st)

    def wait(pv, tv, sp, st):
        pltpu.make_async_copy(pred_hbm.at[0, 0, :, pl.ds(0, BLK)], pv, sp).wait()
        pltpu.make_async_copy(targ_hbm.at[0, 0, :, pl.ds(0, BLK)], tv, st).wait()

    def compute(r, pv, tv, acc):
        def g_loop(g, s):
            return s + _group_loss(pv, tv, g * L)

        s = lax.fori_loop(0, BLK // L, g_loop, jnp.zeros((L,), jnp.float32))
        active = (w + NW * r) < UNITS
        return acc + jnp.where(active, s, 0.0)

    issue(0, pv0, tv0, sp0, st0)

    def body2(kk, acc):
        r0 = 2 * kk
        wait(pv0, tv0, sp0, st0)
        issue(r0 + 1, pv1, tv1, sp1, st1)
        acc = compute(r0, pv0, tv0, acc)
        wait(pv1, tv1, sp1, st1)
        issue(r0 + 2, pv0, tv0, sp0, st0)
        return compute(r0 + 1, pv1, tv1, acc)

    acc = lax.fori_loop(0, (ROUNDS - 1) // 2, body2,
                        jnp.zeros((L,), jnp.float32))
    wait(pv0, tv0, sp0, st0)
    acc = compute(ROUNDS - 1, pv0, tv0, acc)

    acc_v[pl.ds(0, L)] = acc
    for k in range(1, 8):
        acc_v[pl.ds(k * L, L)] = jnp.zeros((L,), jnp.float32)
    pltpu.sync_copy(acc_v, out_hbm.at[w])


@jax.jit
def _sc_loss(pred_t, targ_t):
    mesh = plsc.VectorSubcoreMesh(core_axis_name="c", subcore_axis_name="s")
    fn = pl.kernel(
        _body,
        out_type=jax.ShapeDtypeStruct((NW, 128), jnp.float32),
        mesh=mesh,
        scratch_types=[
            pltpu.VMEM((C, BLK), jnp.float32),
            pltpu.VMEM((C, BLK), jnp.float32),
            pltpu.VMEM((C, BLK), jnp.float32),
            pltpu.VMEM((C, BLK), jnp.float32),
            pltpu.VMEM((128,), jnp.float32),
            pltpu.SemaphoreType.DMA,
            pltpu.SemaphoreType.DMA,
            pltpu.SemaphoreType.DMA,
            pltpu.SemaphoreType.DMA,
        ],
        compiler_params=pltpu.CompilerParams(
            use_tc_tiling_on_sc=True, needs_layout_passes=False),
    )
    return fn(pred_t, targ_t)


def kernel(pred_tensor, target_tensor):
    n = pred_tensor.shape[0]
    pt = jnp.transpose(pred_tensor, (1, 2, 3, 0))
    tt = jnp.transpose(target_tensor, (1, 2, 3, 0))
    partials = _sc_loss(pt, tt)
    return jnp.sum(partials) / jnp.float32(n)


# speedup vs baseline: 1.0027x; 1.0027x over previous
import jax
import jax.numpy as jnp
from jax import lax
from jax.experimental import pallas as pl
from jax.experimental.pallas import tpu as pltpu
from jax.experimental.pallas import tpu_sc as plsc

NC, NS, L = 2, 16, 16
NW = NC * NS
GRID = 14
CELLS = GRID * GRID
C = 30
NB = 256
GROUPS = NB // L


def _sq(x):
    return x * x


def _sqrt16(x):
    i = lax.bitcast_convert_type(x, jnp.int32)
    i = jnp.int32(0x5F3759DF) - lax.shift_right_arithmetic(i, 1)
    y = lax.bitcast_convert_type(i, jnp.float32)
    for _ in range(4):
        y = y * (1.5 - 0.5 * x * y * y)
    return x * y


def _group_loss(pv, tv, b0):
    d = pv[10, pl.ds(b0, L)] - tv[10, pl.ds(b0, L)]
    cls = d * d
    for j in range(11, C):
        d = pv[j, pl.ds(b0, L)] - tv[j, pl.ds(b0, L)]
        cls = cls + d * d

    p = [pv[j, pl.ds(b0, L)] for j in range(10)]
    t = [tv[j, pl.ds(b0, L)] for j in range(10)]
    conf = t[4]
    coo = jnp.where(conf > 0, 1.0, 0.0).astype(jnp.float32)
    noo = jnp.where(conf == 0, 1.0, 0.0).astype(jnp.float32)
    noo_row = _sq(p[4] - t[4]) + _sq(p[9] - t[9])

    t_xmin = t[0] - 0.5 * t[2]
    t_ymin = t[1] - 0.5 * t[3]
    t_xmax = t[0] + 0.5 * t[2]
    t_ymax = t[1] + 0.5 * t[3]
    area2 = t[2] * t[3]

    ious = []
    for k in (0, 5):
        xmin = p[k + 0] - 0.5 * p[k + 2]
        ymin = p[k + 1] - 0.5 * p[k + 3]
        xmax = p[k + 0] + 0.5 * p[k + 2]
        ymax = p[k + 1] + 0.5 * p[k + 3]
        ltx = jnp.maximum(xmin, t_xmin)
        lty = jnp.maximum(ymin, t_ymin)
        rbx = jnp.minimum(xmax, t_xmax)
        rby = jnp.minimum(ymax, t_ymax)
        whx = jnp.where(rbx - ltx < 0, 1.0, 0.0).astype(jnp.float32)
        why = jnp.where(rby - lty < 0, 1.0, 0.0).astype(jnp.float32)
        inter = whx * why
        area1 = p[k + 2] * p[k + 3]
        ious.append(inter / (area1 + area2 - inter))
    iou0, iou1 = ious
    isn0 = iou0 != iou0
    isn1 = iou1 != iou1
    sel = (iou1 > iou0) | (isn1 & (~isn0))

    rp = [jnp.where(sel, p[5 + j], p[j]) for j in range(5)]
    rt = [jnp.where(sel, t[5 + j], t[j]) for j in range(5)]

    contain = _sq(rp[4] - rt[4])
    loc = (_sq(rp[0] - rt[0]) + _sq(rp[1] - rt[1])
           + _sq(_sqrt16(rp[2]) - _sqrt16(rt[2]))
           + _sq(_sqrt16(rp[3]) - _sqrt16(rt[3])))
    return coo * (loc + 2.0 * contain + cls) + noo * noo_row


BLK = 128
UNITS = CELLS * (NB // BLK)
ROUNDS = (UNITS + NW - 1) // NW


def _body(pred_hbm, targ_hbm, out_hbm,
          pv0, tv0, pv1, tv1, acc_v, sp0, st0, sp1, st1):
    w = lax.axis_index("s") * NC + lax.axis_index("c")

    def unit_id(r):
        return jnp.minimum(w + NW * r, UNITS - 1)

    def src_refs(h):
        cell = h // 2
        ci = cell // GRID
        cj = cell - ci * GRID
        b0 = (h - cell * 2) * BLK
        return (pred_hbm.at[ci, cj, :, pl.ds(b0, BLK)],
                targ_hbm.at[ci, cj, :, pl.ds(b0, BLK)])

    def issue(r, pv, tv, sp, st):
        ps, ts = src_refs(unit_id(r))
        pltpu.async_copy(ps, pv, sp)
        pltpu.async_copy(ts, tv, st)

    def wait(pv, tv, sp, st):
        pltpu.make_async_copy(pred_hbm.at[0, 0, :, pl.ds(0, BLK)], pv, sp).wait()
        pltpu.make_async_copy(targ_hbm.at[0, 0, :, pl.ds(0, BLK)], tv, st).wait()

    def compute(r, pv, tv, acc):
        def g_loop(g, s):
            return s + _group_loss(pv, tv, g * L)

        s = lax.fori_loop(0, BLK // L, g_loop, jnp.zeros((L,), jnp.float32))
        active = (w + NW * r) < UNITS
        return acc + jnp.where(active, s, 0.0)

    issue(0, pv0, tv0, sp0, st0)

    def body2(kk, acc):
        r0 = 2 * kk
        wait(pv0, tv0, sp0, st0)
        issue(r0 + 1, pv1, tv1, sp1, st1)
        acc = compute(r0, pv0, tv0, acc)
        wait(pv1, tv1, sp1, st1)
        issue(r0 + 2, pv0, tv0, sp0, st0)
        return compute(r0 + 1, pv1, tv1, acc)

    acc = lax.fori_loop(0, (ROUNDS - 1) // 2, body2,
                        jnp.zeros((L,), jnp.float32))
    wait(pv0, tv0, sp0, st0)
    acc = compute(ROUNDS - 1, pv0, tv0, acc)

    acc_v[pl.ds(0, L)] = acc
    for k in range(1, 8):
        acc_v[pl.ds(k * L, L)] = jnp.zeros((L,), jnp.float32)
    pltpu.sync_copy(acc_v, out_hbm.at[w])


@jax.jit
def _sc_loss(pred_t, targ_t):
    mesh = plsc.VectorSubcoreMesh(core_axis_name="c", subcore_axis_name="s")
    fn = pl.kernel(
        _body,
        out_type=jax.ShapeDtypeStruct((NW, 128), jnp.float32),
        mesh=mesh,
        scratch_types=[
            pltpu.VMEM((C, BLK), jnp.float32),
            pltpu.VMEM((C, BLK), jnp.float32),
            pltpu.VMEM((C, BLK), jnp.float32),
            pltpu.VMEM((C, BLK), jnp.float32),
            pltpu.VMEM((128,), jnp.float32),
            pltpu.SemaphoreType.DMA,
            pltpu.SemaphoreType.DMA,
            pltpu.SemaphoreType.DMA,
            pltpu.SemaphoreType.DMA,
        ],
        compiler_params=pltpu.CompilerParams(
            use_tc_tiling_on_sc=True, needs_layout_passes=False,
            disable_bounds_checks=True, disable_semaphore_checks=True),
    )
    return fn(pred_t, targ_t)


def kernel(pred_tensor, target_tensor):
    n = pred_tensor.shape[0]
    pt = jnp.transpose(pred_tensor, (1, 2, 3, 0))
    tt = jnp.transpose(target_tensor, (1, 2, 3, 0))
    partials = _sc_loss(pt, tt)
    return jnp.sum(partials) / jnp.float32(n)
